# baseline (speedup 1.0000x reference)
import jax
import jax.numpy as jnp
from jax import lax
from jax.experimental import pallas as pl
from jax.experimental.pallas import tpu as pltpu
from jax.experimental.pallas import tpu_sc as plsc

B = 4
N = 4096
D = 2048
L = 16

NC = 2
NS = 16
NW = NC * NS

C = 128
K = 1
SC_COLS = C * K
TC_COLS = D - SC_COLS
H = 2
NH = N // H
R = 256
NCHUNK = NH // R
G = C // L

TCB = 384


def _sc_body(x_hbm, outv_hbm, outi_hbm, buf0, buf1, vstage, istage,
             sem0, sem1):
    wid = lax.axis_index("s") * NC + lax.axis_index("c")

    bufs = (buf0, buf1)
    sems = (sem0, sem1)

    b = wid // (K * H)
    blk = (wid // H) % K
    h = wid % H
    c0 = blk * C
    row_base = b * N + h * NH

    def start(chunk, k):
        pltpu.async_copy(
            x_hbm.at[pl.ds(row_base + chunk * R, R), pl.ds(c0, C)],
            bufs[k], sems[k])

    def start2(chunk, k):
        chunk = lax.min(chunk, NCHUNK - 1)
        start(chunk, k)

    def wait(k):
        pltpu.make_async_copy(
            x_hbm.at[pl.ds(row_base, R), pl.ds(c0, C)],
            bufs[k], sems[k]).wait()

    ones = jnp.ones((L,), jnp.int32)

    def rows(buf, carry):
        def row_body(r, carry):
            mvs, mis, ridx = carry
            mvs = list(mvs)
            mis = list(mis)
            for g in range(G):
                xv = buf[r, pl.ds(g * L, L)]
                m = xv < mvs[g]
                mvs[g] = jnp.where(m, xv, mvs[g])
                mis[g] = jnp.where(m, ridx, mis[g])
            return tuple(mvs), tuple(mis), ridx + ones

        return lax.fori_loop(0, R, row_body, carry)

    carry = (
        tuple(jnp.full((L,), jnp.inf, jnp.float32) for _ in range(G)),
        tuple(jnp.zeros((L,), jnp.int32) for _ in range(G)),
        jnp.full((L,), h * NH, jnp.int32),
    )

    start(0, 0)

    def chunk_body(cidx, carry):
        start2(2 * cidx + 1, 1)
        wait(0)
        carry = rows(bufs[0], carry)
        start2(2 * cidx + 2, 0)
        wait(1)
        return rows(bufs[1], carry)

    carry = lax.fori_loop(0, NCHUNK // 2, chunk_body, carry)

    minvs, minis, _ = carry
    for g in range(G):
        vstage[pl.ds(g * L, L)] = minvs[g]
        istage[pl.ds(g * L, L)] = minis[g]
    obase = h * (B * SC_COLS) + b * SC_COLS + c0
    pltpu.sync_copy(vstage, outv_hbm.at[pl.ds(obase, C)])
    pltpu.sync_copy(istage, outi_hbm.at[pl.ds(obase, C)])


def _tc_body(x_ref, o_ref):
    v = x_ref[0]
    m = jnp.min(v, axis=0)
    iota = lax.broadcasted_iota(jnp.int32, (N, TCB), 0)
    masked = jnp.where(v == m[None, :], iota, jnp.int32(N))
    o_ref[0, 0] = jnp.min(masked, axis=0)


@jax.jit
def kernel(x):
    x2 = x.reshape(B * N, D)
    tc_out = pl.pallas_call(
        _tc_body,
        grid=(B, TC_COLS // TCB),
        in_specs=[pl.BlockSpec(
            (1, N, TCB), lambda b, j: (b, 0, j + SC_COLS // TCB))],
        out_specs=pl.BlockSpec((1, 1, TCB), lambda b, j: (b, 0, j)),
        out_shape=jax.ShapeDtypeStruct((B, 1, TC_COLS), jnp.int32),
    )(x)
    tc_out = tc_out.reshape(B, TC_COLS)

    mesh = plsc.VectorSubcoreMesh(core_axis_name="c", subcore_axis_name="s")
    sc_v, sc_i = pl.kernel(
        _sc_body,
        out_type=(
            jax.ShapeDtypeStruct((H * B * SC_COLS,), jnp.float32),
            jax.ShapeDtypeStruct((H * B * SC_COLS,), jnp.int32),
        ),
        mesh=mesh,
        scratch_types=[
            pltpu.VMEM((R, C), jnp.float32),
            pltpu.VMEM((R, C), jnp.float32),
            pltpu.VMEM((C,), jnp.float32),
            pltpu.VMEM((C,), jnp.int32),
            pltpu.SemaphoreType.DMA,
            pltpu.SemaphoreType.DMA,
        ],
    )(x2)

    pv = sc_v.reshape(H, B, SC_COLS)
    pi = sc_i.reshape(H, B, SC_COLS)
    take0 = pv[0] <= pv[1]
    sc_out = jnp.where(take0, pi[0], pi[1])

    out = jnp.concatenate([sc_out, tc_out], axis=1)
    return out.astype(jnp.int64)


# speedup vs baseline: 1.1800x; 1.1800x over previous
import jax
import jax.numpy as jnp
from jax import lax
from jax.experimental import pallas as pl
from jax.experimental.pallas import tpu as pltpu
from jax.experimental.pallas import tpu_sc as plsc

B = 4
N = 4096
D = 2048
L = 16

NC = 2
NS = 16
NW = NC * NS

SC_ROWS = 1536
TC_ROWS = N - SC_ROWS
SC_ROW0 = TC_ROWS

C = 128
NBLK = D // C
R = 256
NCHUNK = SC_ROWS // R
G = C // L

RB = 512
NRB = TC_ROWS // RB


def _sc_body(x_hbm, outv_hbm, outi_hbm, buf0, buf1, vstage, istage,
             sem0, sem1):
    wid = lax.axis_index("s") * NC + lax.axis_index("c")

    bufs = (buf0, buf1)
    sems = (sem0, sem1)

    blk = wid % NBLK
    c0 = blk * C
    ones = jnp.ones((L,), jnp.int32)

    for t in range(2):
        b = wid // NBLK + 2 * t
        row_base = b * N + SC_ROW0

        def start(chunk, k):
            pltpu.async_copy(
                x_hbm.at[pl.ds(row_base + chunk * R, R), pl.ds(c0, C)],
                bufs[k], sems[k])

        def wait(k):
            pltpu.make_async_copy(
                x_hbm.at[pl.ds(row_base, R), pl.ds(c0, C)],
                bufs[k], sems[k]).wait()

        def rows(buf, carry):
            def row_body(r, carry):
                mvs, mis, ridx = carry
                mvs = list(mvs)
                mis = list(mis)
                for g in range(G):
                    xv = buf[r, pl.ds(g * L, L)]
                    m = xv < mvs[g]
                    mvs[g] = jnp.where(m, xv, mvs[g])
                    mis[g] = jnp.where(m, ridx, mis[g])
                return tuple(mvs), tuple(mis), ridx + ones

            return lax.fori_loop(0, R, row_body, carry)

        carry = (
            tuple(jnp.full((L,), jnp.inf, jnp.float32) for _ in range(G)),
            tuple(jnp.zeros((L,), jnp.int32) for _ in range(G)),
            jnp.full((L,), SC_ROW0, jnp.int32),
        )

        start(0, 0)
        for i in range(NCHUNK):
            if i + 1 < NCHUNK:
                start(i + 1, (i + 1) % 2)
            wait(i % 2)
            carry = rows(bufs[i % 2], carry)

        minvs, minis, _ = carry
        for g in range(G):
            vstage[pl.ds(g * L, L)] = minvs[g]
            istage[pl.ds(g * L, L)] = minis[g]
        obase = b * D + c0
        pltpu.sync_copy(vstage, outv_hbm.at[pl.ds(obase, C)])
        pltpu.sync_copy(istage, outi_hbm.at[pl.ds(obase, C)])


def _tc_body(x_ref, ov_ref, oi_ref, mv_s, mi_s):
    r = pl.program_id(1)
    v = x_ref[0]
    m = jnp.min(v, axis=0)
    iota = lax.broadcasted_iota(jnp.int32, (RB, D), 0) + r * RB
    idx = jnp.min(jnp.where(v == m[None, :], iota, jnp.int32(N)), axis=0)

    @pl.when(r == 0)
    def _():
        mv_s[0] = m
        mi_s[0] = idx

    @pl.when(r > 0)
    def _():
        prev = mv_s[0]
        upd = m < prev
        mv_s[0] = jnp.where(upd, m, prev)
        mi_s[0] = jnp.where(upd, idx, mi_s[0])

    @pl.when(r == NRB - 1)
    def _():
        ov_ref[0, 0] = mv_s[0]
        oi_ref[0, 0] = mi_s[0]


@jax.jit
def kernel(x):
    x2 = x.reshape(B * N, D)

    tc_v, tc_i = pl.pallas_call(
        _tc_body,
        grid=(B, NRB),
        in_specs=[pl.BlockSpec((1, RB, D), lambda b, r: (b, r, 0))],
        out_specs=[
            pl.BlockSpec((1, 1, D), lambda b, r: (b, 0, 0)),
            pl.BlockSpec((1, 1, D), lambda b, r: (b, 0, 0)),
        ],
        out_shape=[
            jax.ShapeDtypeStruct((B, 1, D), jnp.float32),
            jax.ShapeDtypeStruct((B, 1, D), jnp.int32),
        ],
        scratch_shapes=[
            pltpu.VMEM((1, D), jnp.float32),
            pltpu.VMEM((1, D), jnp.int32),
        ],
    )(x)

    mesh = plsc.VectorSubcoreMesh(core_axis_name="c", subcore_axis_name="s")
    sc_v, sc_i = pl.kernel(
        _sc_body,
        out_type=(
            jax.ShapeDtypeStruct((B * D,), jnp.float32),
            jax.ShapeDtypeStruct((B * D,), jnp.int32),
        ),
        mesh=mesh,
        scratch_types=[
            pltpu.VMEM((R, C), jnp.float32),
            pltpu.VMEM((R, C), jnp.float32),
            pltpu.VMEM((C,), jnp.float32),
            pltpu.VMEM((C,), jnp.int32),
            pltpu.SemaphoreType.DMA,
            pltpu.SemaphoreType.DMA,
        ],
    )(x2)

    mv_tc = tc_v.reshape(B, D)
    mi_tc = tc_i.reshape(B, D)
    mv_sc = sc_v.reshape(B, D)
    mi_sc = sc_i.reshape(B, D)
    out = jnp.where(mv_tc <= mv_sc, mi_tc, mi_sc)
    return out.astype(jnp.int64)


# speedup vs baseline: 1.2160x; 1.0304x over previous
import jax
import jax.numpy as jnp
from jax import lax
from jax.experimental import pallas as pl
from jax.experimental.pallas import tpu as pltpu
from jax.experimental.pallas import tpu_sc as plsc

B = 4
N = 4096
D = 2048
L = 16

NC = 2
NS = 16
NW = NC * NS

SC_ROWS = 1536
TC_ROWS = N - SC_ROWS
SC_ROW0 = TC_ROWS

C = 128
NBLK = D // C
R = 256
NCHUNK = SC_ROWS // R
G = C // L

RB = 512
NRB = TC_ROWS // RB


def _sc_body(x_hbm, outv_hbm, outi_hbm, buf0, buf1, vstage, istage,
             sem0, sem1):
    wid = lax.axis_index("s") * NC + lax.axis_index("c")

    bufs = (buf0, buf1)
    sems = (sem0, sem1)

    blk = wid % NBLK
    c0 = blk * C
    ones = jnp.ones((L,), jnp.int32)

    for t in range(2):
        b = wid // NBLK + 2 * t
        row_base = b * N + SC_ROW0

        def start(chunk, k):
            pltpu.async_copy(
                x_hbm.at[pl.ds(row_base + chunk * R, R), pl.ds(c0, C)],
                bufs[k], sems[k])

        def wait(k):
            pltpu.make_async_copy(
                x_hbm.at[pl.ds(row_base, R), pl.ds(c0, C)],
                bufs[k], sems[k]).wait()

        def rows(buf, carry):
            def row_body(r, carry):
                mvs, mis, ridx = carry
                mvs = list(mvs)
                mis = list(mis)
                for g in range(G):
                    xv = buf[r, pl.ds(g * L, L)]
                    m = xv < mvs[g]
                    mvs[g] = jnp.where(m, xv, mvs[g])
                    mis[g] = jnp.where(m, ridx, mis[g])
                return tuple(mvs), tuple(mis), ridx + ones

            return lax.fori_loop(0, R, row_body, carry)

        carry = (
            tuple(jnp.full((L,), jnp.inf, jnp.float32) for _ in range(G)),
            tuple(jnp.zeros((L,), jnp.int32) for _ in range(G)),
            jnp.full((L,), SC_ROW0, jnp.int32),
        )

        start(0, 0)
        for i in range(NCHUNK):
            if i + 1 < NCHUNK:
                start(i + 1, (i + 1) % 2)
            wait(i % 2)
            carry = rows(bufs[i % 2], carry)

        minvs, minis, _ = carry
        for g in range(G):
            vstage[pl.ds(g * L, L)] = minvs[g]
            istage[pl.ds(g * L, L)] = minis[g]
        obase = b * D + c0
        pltpu.sync_copy(vstage, outv_hbm.at[pl.ds(obase, C)])
        pltpu.sync_copy(istage, outi_hbm.at[pl.ds(obase, C)])


def _tc_body(x_ref, ov_ref, oi_ref, mv_s, mi_s):
    r = pl.program_id(1)
    v = x_ref[0]
    m = jnp.min(v, axis=0)
    iota = lax.broadcasted_iota(jnp.int32, (RB, D), 0) + r * RB
    idx = jnp.min(jnp.where(v == m[None, :], iota, jnp.int32(N)), axis=0)

    @pl.when(r == 0)
    def _():
        mv_s[0] = m
        mi_s[0] = idx

    @pl.when(r > 0)
    def _():
        prev = mv_s[0]
        upd = m < prev
        mv_s[0] = jnp.where(upd, m, prev)
        mi_s[0] = jnp.where(upd, idx, mi_s[0])

    @pl.when(r == NRB - 1)
    def _():
        ov_ref[...] = mv_s[0]
        oi_ref[...] = mi_s[0]


@jax.jit
def kernel(x):
    x2 = x.reshape(B * N, D)

    tc_v, tc_i = pl.pallas_call(
        _tc_body,
        grid=(B, NRB),
        in_specs=[pl.BlockSpec((1, RB, D), lambda b, r: (b, r, 0))],
        out_specs=[
            pl.BlockSpec((D,), lambda b, r: (b,)),
            pl.BlockSpec((D,), lambda b, r: (b,)),
        ],
        out_shape=[
            jax.ShapeDtypeStruct((B * D,), jnp.float32),
            jax.ShapeDtypeStruct((B * D,), jnp.int32),
        ],
        scratch_shapes=[
            pltpu.VMEM((1, D), jnp.float32),
            pltpu.VMEM((1, D), jnp.int32),
        ],
    )(x)

    mesh = plsc.VectorSubcoreMesh(core_axis_name="c", subcore_axis_name="s")
    sc_v, sc_i = pl.kernel(
        _sc_body,
        out_type=(
            jax.ShapeDtypeStruct((B * D,), jnp.float32),
            jax.ShapeDtypeStruct((B * D,), jnp.int32),
        ),
        mesh=mesh,
        scratch_types=[
            pltpu.VMEM((R, C), jnp.float32),
            pltpu.VMEM((R, C), jnp.float32),
            pltpu.VMEM((C,), jnp.float32),
            pltpu.VMEM((C,), jnp.int32),
            pltpu.SemaphoreType.DMA,
            pltpu.SemaphoreType.DMA,
        ],
    )(x2)

    out = jnp.where(tc_v <= sc_v, tc_i, sc_i)
    return out.astype(jnp.int64).reshape(B, D)
